# baseline (speedup 1.0000x reference)
import functools

import jax
import jax.numpy as jnp
from jax import lax
from jax.experimental import pallas as pl
from jax.experimental.pallas import tpu as pltpu
from jax.experimental.pallas import tpu_sc as plsc

VOCAB = 1000000
SEQ = 200
HID = 64
BATCH = 4096
EPS = 1e-12

NW = 32
BPW = BATCH // NW
RPW = BPW * SEQ
SUP = 2 * SEQ
NSUP = RPW // SUP
SUBS = 4
GCH = SUP // SUBS
GROUPS = SUP // 16
KUNROLL = 8


def _rsqrt(v):
    i = plsc.bitcast(v, jnp.int32)
    i = jnp.int32(0x5F3759DF) - (i >> 1)
    y = plsc.bitcast(i, jnp.float32)
    for _ in range(3):
        y = y * (1.5 - 0.5 * v * y * y)
    return y


def _make_emb_kernel():
    mesh = plsc.VectorSubcoreMesh(core_axis_name="c", subcore_axis_name="s")

    @functools.partial(
        pl.kernel,
        mesh=mesh,
        compiler_params=pltpu.CompilerParams(
            needs_layout_passes=False, use_tc_tiling_on_sc=False),
        out_type=jax.ShapeDtypeStruct((BATCH, SEQ, HID), jnp.float32),
        scratch_types=[
            pltpu.VMEM((NSUP, SUBS, GCH), jnp.int32),
            pltpu.VMEM((2, SEQ, HID), jnp.float32),
            pltpu.VMEM((2, SEQ, HID), jnp.float32),
            pltpu.VMEM((SEQ, HID), jnp.float32),
            pltpu.VMEM((HID * 16,), jnp.float32),
            pltpu.VMEM((HID * 16,), jnp.float32),
            pltpu.SemaphoreType.DMA,
            pltpu.SemaphoreType.DMA,
            pltpu.SemaphoreType.DMA,
            pltpu.SemaphoreType.DMA,
        ],
    )
    def emb(state_hbm, table_hbm, pos_hbm, gamma_hbm, beta_hbm, out_hbm,
            idx_v, rows0, rows1, pos_v, gamma_v, beta_v, gs0, gs1, os0, os1):
        rows = (rows0, rows1)
        gsem = (gs0, gs1)
        osem = (os0, os1)
        wid = lax.axis_index("s") * 2 + lax.axis_index("c")
        pltpu.sync_copy(state_hbm.at[wid], idx_v)
        pltpu.sync_copy(pos_hbm, pos_v)
        pltpu.sync_copy(gamma_hbm, gamma_v)
        pltpu.sync_copy(beta_hbm, beta_v)
        base_b = wid * BPW
        lanes = lax.iota(jnp.int32, 16)

        def fire_gather(c, b):
            for j in range(SUBS):
                pltpu.async_copy(
                    table_hbm.at[idx_v.at[c, j]],
                    rows[b].at[j // 2, pl.ds((j % 2) * GCH, GCH)], gsem[b])

        def wait_gather(b):
            pltpu.make_async_copy(
                out_hbm.at[pl.ds(0, 2)], rows[b], gsem[b]).wait()

        def fire_out(c, b):
            pltpu.async_copy(
                rows[b], out_hbm.at[pl.ds(base_b + 2 * c, 2)], osem[b])

        def wait_out(b):
            pltpu.make_async_copy(
                rows[b], out_hbm.at[pl.ds(0, 2)], osem[b]).wait()

        def compute(c, b):
            buf = rows[b]

            def group_body(gi, _):
                lr = lanes + gi * 16
                q = (lr >= SEQ).astype(jnp.int32)
                pvec = lr - q * SEQ

                def p1(k8, carry):
                    sa, sb, s2a, s2b = carry
                    for kk in range(KUNROLL):
                        hvec = (lanes + k8 * KUNROLL + kk) & (HID - 1)
                        t = plsc.load_gather(buf, [q, pvec, hvec])
                        p = plsc.load_gather(pos_v, [pvec, hvec])
                        x = t + p
                        plsc.store_scatter(buf, [q, pvec, hvec], x)
                        if kk % 2 == 0:
                            sa = sa + x
                            s2a = s2a + x * x
                        else:
                            sb = sb + x
                            s2b = s2b + x * x
                    return sa, sb, s2a, s2b

                zero = jnp.zeros((16,), jnp.float32)
                sa, sb, s2a, s2b = lax.fori_loop(
                    0, HID // KUNROLL, p1, (zero, zero, zero, zero))
                mean = (sa + sb) * (1.0 / HID)
                var = (s2a + s2b) * (1.0 / HID) - mean * mean
                rstd = _rsqrt(var + EPS)

                def p2(k8, carry):
                    for kk in range(KUNROLL):
                        k = k8 * KUNROLL + kk
                        hvec = (lanes + k) & (HID - 1)
                        x = plsc.load_gather(buf, [q, pvec, hvec])
                        gam = gamma_v[pl.ds(k * 16, 16)]
                        bet = beta_v[pl.ds(k * 16, 16)]
                        y = (x - mean) * rstd * gam + bet
                        plsc.store_scatter(buf, [q, pvec, hvec], y)
                    return carry

                lax.fori_loop(0, HID // KUNROLL, p2, 0)
                return 0

            lax.fori_loop(0, GROUPS, group_body, 0)

        fire_gather(0, 0)
        fire_gather(1, 1)
        wait_gather(0)
        compute(0, 0)
        fire_out(0, 0)

        def pair_body(i, _):
            c2 = 1 + 2 * i
            wait_gather(1)
            wait_out(0)
            fire_gather(c2 + 1, 0)
            compute(c2, 1)
            fire_out(c2, 1)
            wait_gather(0)
            wait_out(1)
            fire_gather(c2 + 2, 1)
            compute(c2 + 1, 0)
            fire_out(c2 + 1, 0)
            return 0

        lax.fori_loop(0, (NSUP - 2) // 2, pair_body, 0)
        wait_gather(1)
        compute(NSUP - 1, 1)
        fire_out(NSUP - 1, 1)
        wait_out(0)
        wait_out(1)

    return emb


_emb_kernel = _make_emb_kernel()


def kernel(state, token_table, pos_table, ln_gamma, ln_beta):
    state_w = state.reshape(NW, NSUP, SUBS, GCH)
    rot = (jnp.arange(HID)[:, None] + jnp.arange(16)[None, :]) % HID
    gamma_rot = ln_gamma[rot].reshape(-1)
    beta_rot = ln_beta[rot].reshape(-1)
    return _emb_kernel(state_w, token_table, pos_table, gamma_rot, beta_rot)


# speedup vs baseline: 1.2288x; 1.2288x over previous
import functools

import jax
import jax.numpy as jnp
from jax import lax
from jax.experimental import pallas as pl
from jax.experimental.pallas import tpu as pltpu
from jax.experimental.pallas import tpu_sc as plsc

VOCAB = 1000000
SEQ = 200
HID = 64
BATCH = 4096
EPS = 1e-12

NW = 32
BPW = BATCH // NW
RPW = BPW * SEQ
SPC = 2
SUP = SPC * BPW
NSUP = SEQ // SPC
GROUPS = SUP // 16
KUNROLL = 8


def _rsqrt(v):
    i = plsc.bitcast(v, jnp.int32)
    i = jnp.int32(0x5F3759DF) - (i >> 1)
    y = plsc.bitcast(i, jnp.float32)
    for _ in range(3):
        y = y * (1.5 - 0.5 * v * y * y)
    return y


def _make_emb_kernel():
    mesh = plsc.VectorSubcoreMesh(core_axis_name="c", subcore_axis_name="s")

    @functools.partial(
        pl.kernel,
        mesh=mesh,
        compiler_params=pltpu.CompilerParams(
            needs_layout_passes=False, use_tc_tiling_on_sc=True),
        out_type=jax.ShapeDtypeStruct((SEQ, 8, NW, 8, 128), jnp.float32),
        scratch_types=[
            pltpu.VMEM((SUP, 128), jnp.float32),
            pltpu.VMEM((SUP, 128), jnp.float32),
            pltpu.VMEM((SPC, 8, 8, 128), jnp.float32),
            pltpu.VMEM((SPC, 8, 8, 128), jnp.float32),
            pltpu.VMEM((SUP,), jnp.int32),
            pltpu.VMEM((SUP,), jnp.int32),
            pltpu.VMEM((SUP,), jnp.int32),
            pltpu.VMEM((SUP,), jnp.int32),
            pltpu.VMEM((SEQ * HID,), jnp.float32),
            pltpu.VMEM((HID * 16,), jnp.float32),
            pltpu.VMEM((HID * 16,), jnp.float32),
            pltpu.SemaphoreType.DMA,
            pltpu.SemaphoreType.DMA,
            pltpu.SemaphoreType.DMA,
            pltpu.SemaphoreType.DMA,
        ],
    )
    def emb(state_hbm, table_hbm, pos_hbm, gamma_hbm, beta_hbm, out_hbm,
            rows0, rows1, y0, y1, ib0, ib1, ih0, ih1,
            pos_v, gamma_v, beta_v, gs0, gs1, os0, os1):
        rows = (rows0, rows1)
        ybuf = (y0, y1)
        ibuf = (ib0, ib1)
        hbuf = (ih0, ih1)
        gsem = (gs0, gs1)
        osem = (os0, os1)
        wid = lax.axis_index("s") * 2 + lax.axis_index("c")
        pltpu.sync_copy(pos_hbm, pos_v)
        pltpu.sync_copy(gamma_hbm, gamma_v)
        pltpu.sync_copy(beta_hbm, beta_v)
        lanes = lax.iota(jnp.int32, 16)

        def fire_gather(c, b):
            pltpu.sync_copy(state_hbm.at[wid, pl.ds(c * SUP, SUP)], ibuf[b])
            for m in range(SUP // 16):
                iv = ibuf[b][pl.ds(m * 16, 16)]
                hbuf[b][pl.ds(m * 16, 16)] = iv >> 1
            for j in range(SPC):
                pltpu.async_copy(
                    table_hbm.at[hbuf[b].at[pl.ds(j * 128, 128)]],
                    rows[b].at[pl.ds(j * 128, 128)], gsem[b])

        def wait_gather(b):
            pltpu.make_async_copy(
                table_hbm.at[pl.ds(0, SUP)], rows[b], gsem[b]).wait()

        def fire_out(c, b):
            pltpu.async_copy(
                ybuf[b], out_hbm.at[pl.ds(c * SPC, SPC), :, wid], osem[b])

        def wait_out(b):
            pltpu.make_async_copy(
                ybuf[b], out_hbm.at[pl.ds(0, SPC), :, wid], osem[b]).wait()

        def compute(c, b):
            buf = rows[b]

            def group_body(gi, _):
                s_i = gi // 8
                b0 = (gi % 8) * 16
                rr = s_i * 128 + b0 + lanes
                pbase = (c * SPC + s_i) * HID
                par = plsc.load_gather(ibuf[b], [rr]) & 1
                par64 = par << 6
                bl = b0 + lanes
                si_v = jnp.full((16,), s_i, jnp.int32)

                def p1(k8, carry):
                    sa, sb, s2a, s2b = carry
                    for kk in range(KUNROLL):
                        hvec = (lanes + k8 * KUNROLL + kk) & (HID - 1)
                        col = par64 + hvec
                        t = plsc.load_gather(buf, [rr, col])
                        p = plsc.load_gather(pos_v, [pbase + hvec])
                        x = t + p
                        plsc.store_scatter(buf, [rr, col], x)
                        if kk % 2 == 0:
                            sa = sa + x
                            s2a = s2a + x * x
                        else:
                            sb = sb + x
                            s2b = s2b + x * x
                    return sa, sb, s2a, s2b

                zero = jnp.zeros((16,), jnp.float32)
                sa, sb, s2a, s2b = lax.fori_loop(
                    0, HID // KUNROLL, p1, (zero, zero, zero, zero))
                mean = (sa + sb) * (1.0 / HID)
                var = (s2a + s2b) * (1.0 / HID) - mean * mean
                rstd = _rsqrt(var + EPS)

                def p2(k8, carry):
                    for kk in range(KUNROLL):
                        k = k8 * KUNROLL + kk
                        hvec = (lanes + k) & (HID - 1)
                        col = par64 + hvec
                        x = plsc.load_gather(buf, [rr, col])
                        gam = gamma_v[pl.ds(k * 16, 16)]
                        bet = beta_v[pl.ds(k * 16, 16)]
                        y = (x - mean) * rstd * gam + bet
                        plsc.store_scatter(
                            ybuf[b], [si_v, hvec >> 3, hvec & 7, bl], y)
                    return carry

                lax.fori_loop(0, HID // KUNROLL, p2, 0)
                return 0

            lax.fori_loop(0, GROUPS, group_body, 0)

        fire_gather(0, 0)
        fire_gather(1, 1)
        wait_gather(0)
        compute(0, 0)
        fire_out(0, 0)
        wait_gather(1)
        fire_gather(2, 0)
        compute(1, 1)
        fire_out(1, 1)
        wait_gather(0)
        wait_out(0)
        fire_gather(3, 1)
        compute(2, 0)
        fire_out(2, 0)

        def pair_body(i, _):
            c2 = 3 + 2 * i
            wait_gather(1)
            wait_out(1)
            fire_gather(c2 + 1, 0)
            compute(c2, 1)
            fire_out(c2, 1)
            wait_gather(0)
            wait_out(0)
            fire_gather(c2 + 2, 1)
            compute(c2 + 1, 0)
            fire_out(c2 + 1, 0)
            return 0

        lax.fori_loop(0, (NSUP - 4) // 2, pair_body, 0)
        wait_gather(1)
        wait_out(1)
        compute(NSUP - 1, 1)
        fire_out(NSUP - 1, 1)
        wait_out(0)
        wait_out(1)

    return emb


_emb_kernel = _make_emb_kernel()


def kernel(state, token_table, pos_table, ln_gamma, ln_beta):
    table2 = token_table.reshape(VOCAB // 2, 128)
    state_t = state.reshape(NW, BPW, SEQ).transpose(0, 2, 1).reshape(NW, RPW)
    pos_flat = pos_table.reshape(-1)
    rot = (jnp.arange(HID)[:, None] + jnp.arange(16)[None, :]) % HID
    gamma_rot = ln_gamma[rot].reshape(-1)
    beta_rot = ln_beta[rot].reshape(-1)
    out5 = _emb_kernel(state_t, table2, pos_flat, gamma_rot, beta_rot)
    return out5.transpose(2, 4, 0, 1, 3).reshape(BATCH, SEQ, HID)
